# Initial kernel scaffold; baseline (speedup 1.0000x reference)
import jax
import jax.numpy as jnp
from jax.experimental import pallas as pl


def kernel(phase, tables):
    raise NotImplementedError("write your pallas kernel here")


# speedup vs baseline: 12.7294x; 12.7294x over previous
import functools

import numpy as np
import jax
import jax.numpy as jnp
from jax import lax
from jax.experimental import pallas as pl
from jax.experimental.pallas import tpu as pltpu
from jax.experimental.pallas import tpu_sc as plsc

_F, _V, _D = 26, 100000, 32
_N = 64
_G = (_N * _F) // 128
_LANES = 16


def _worker_count():
    try:
        info = plsc.get_sparse_core_info()
        return info.num_cores, info.num_subcores
    except Exception:
        return 2, 16


@functools.lru_cache(maxsize=None)
def _build(rows_total):
    nc, ns = _worker_count()
    nw = nc * ns
    rows_per_w = rows_total // nw
    nch = rows_per_w // _N

    mesh = plsc.VectorSubcoreMesh(core_axis_name="c", subcore_axis_name="s")

    @functools.partial(
        pl.kernel,
        mesh=mesh,
        compiler_params=pltpu.CompilerParams(use_tc_tiling_on_sc=False),
        out_type=jax.ShapeDtypeStruct((rows_total, _D), jnp.float32),
        scratch_types=[
            pltpu.VMEM((2, _G, 128), jnp.int32),
            pltpu.VMEM((_G, 128), jnp.int32),
            pltpu.VMEM((2, _N * _F, _D), jnp.float32),
            pltpu.VMEM((2, _N, _D), jnp.float32),
            pltpu.SemaphoreType.DMA,
            pltpu.SemaphoreType.DMA,
            pltpu.SemaphoreType.DMA,
            pltpu.SemaphoreType.DMA,
        ],
    )
    def sc_kernel(phase_hbm, offs_hbm, tab_hbm, out_hbm,
                  idx_v, offs_v, rows_v, out_v, g0, g1, s0, s1):
        gsem = (g0, g1)
        osem = (s0, s1)
        wid = lax.axis_index("s") * nc + lax.axis_index("c")
        w_idx_row = wid * (nch * _G)
        w_out_row = wid * rows_per_w

        pltpu.sync_copy(offs_hbm, offs_v)

        def fire(ch, b):
            pltpu.sync_copy(
                phase_hbm.at[pl.ds(w_idx_row + ch * _G, _G)], idx_v.at[b])
            for j in range(_G):
                for k in range(128 // _LANES):
                    sl = pl.ds(k * _LANES, _LANES)
                    idx_v[b, j, sl] = idx_v[b, j, sl] + offs_v[j, sl]
            for j in range(_G):
                pltpu.async_copy(
                    tab_hbm.at[idx_v.at[b, j]],
                    rows_v.at[b, pl.ds(j * 128, 128)],
                    gsem[b])

        def drain_gather(b):
            pltpu.make_async_copy(
                tab_hbm.at[pl.ds(0, _N * _F)], rows_v.at[b], gsem[b]).wait()

        def drain_out(b):
            pltpu.make_async_copy(
                out_v.at[b], out_hbm.at[pl.ds(0, _N)], osem[b]).wait()

        def accum(b):
            def body(n, carry):
                rbase = n * _F
                lo = pl.ds(0, _LANES)
                hi = pl.ds(_LANES, _LANES)
                acc0 = rows_v[b, rbase, lo]
                acc1 = rows_v[b, rbase, hi]
                for f in range(1, _F):
                    acc0 = acc0 + rows_v[b, rbase + f, lo]
                    acc1 = acc1 + rows_v[b, rbase + f, hi]
                out_v[b, n, lo] = acc0
                out_v[b, n, hi] = acc1
                return carry
            lax.fori_loop(0, _N, body, 0)

        fire(0, 0)

        def outer(g, carry):
            for b in range(2):
                ch = g * 2 + b

                @pl.when(ch + 1 < nch)
                def _():
                    fire(ch + 1, 1 - b)

                drain_gather(b)

                @pl.when(ch >= 2)
                def _():
                    drain_out(b)

                accum(b)
                pltpu.async_copy(
                    out_v.at[b],
                    out_hbm.at[pl.ds(w_out_row + ch * _N, _N)],
                    osem[b])
            return carry

        lax.fori_loop(0, nch // 2, outer, 0)
        drain_out(0)
        drain_out(1)

    return sc_kernel


def kernel(phase, tables):
    phase = phase.astype(jnp.int32)
    b_, l_ = phase.shape[0], phase.shape[1]
    rows_total = b_ * l_
    phase2d = phase.reshape(-1, 128)
    tab2d = tables.reshape(_F * _V, _D)
    offs = jnp.asarray(
        np.tile(np.arange(_F, dtype=np.int32) * _V, _N).reshape(_G, 128))
    out = _build(rows_total)(phase2d, offs, tab2d)
    return out.reshape(b_, l_, _D)


# speedup vs baseline: 12.7828x; 1.0042x over previous
import functools

import jax
import jax.numpy as jnp
from jax import lax
from jax.experimental import pallas as pl
from jax.experimental.pallas import tpu as pltpu
from jax.experimental.pallas import tpu_sc as plsc

_F, _V, _D = 26, 100000, 32
_N = 64
_G = (_N * _F) // 128
_LANES = 16


def _worker_count():
    try:
        info = plsc.get_sparse_core_info()
        return info.num_cores, info.num_subcores
    except Exception:
        return 2, 16


@functools.lru_cache(maxsize=None)
def _build(rows_total):
    nc, ns = _worker_count()
    nw = nc * ns
    rows_per_w = rows_total // nw
    nch = rows_per_w // _N

    mesh = plsc.VectorSubcoreMesh(core_axis_name="c", subcore_axis_name="s")

    @functools.partial(
        pl.kernel,
        mesh=mesh,
        compiler_params=pltpu.CompilerParams(
            use_tc_tiling_on_sc=False, needs_layout_passes=False),
        out_type=jax.ShapeDtypeStruct((rows_total, _D), jnp.float32),
        scratch_types=[
            pltpu.VMEM((2, _G, 128), jnp.int32),
            pltpu.VMEM((2, _F, _N), jnp.int32),
            pltpu.VMEM((2, _F * _N, _D), jnp.float32),
            pltpu.VMEM((2, _N, _D), jnp.float32),
            pltpu.SemaphoreType.DMA,
            pltpu.SemaphoreType.DMA,
            pltpu.SemaphoreType.DMA,
            pltpu.SemaphoreType.DMA,
        ],
    )
    def sc_kernel(phase_hbm, tab_hbm, out_hbm,
                  idx_v, pf_v, rows_v, out_v, g0, g1, s0, s1):
        gsem = (g0, g1)
        osem = (s0, s1)
        wid = lax.axis_index("s") * nc + lax.axis_index("c")
        w_idx_row = wid * (nch * _G)
        w_out_row = wid * rows_per_w

        lane = lax.iota(jnp.int32, _LANES)

        def fire(ch, b):
            pltpu.sync_copy(
                phase_hbm.at[pl.ds(w_idx_row + ch * _G, _G)], idx_v.at[b])
            for k in range(_N // _LANES):
                tok26 = (lane + k * _LANES) * _F
                for f in range(_F):
                    pos = tok26 + f
                    row = lax.shift_right_logical(pos, 7)
                    col = lax.bitwise_and(pos, 127)
                    vals = plsc.load_gather(idx_v.at[b], [row, col])
                    pf_v[b, f, pl.ds(k * _LANES, _LANES)] = vals
            for f in range(_F):
                pltpu.async_copy(
                    tab_hbm.at[f].at[pf_v.at[b, f]],
                    rows_v.at[b, pl.ds(f * _N, _N)],
                    gsem[b])

        def drain_gather(b):
            pltpu.make_async_copy(
                tab_hbm.at[0].at[pl.ds(0, _N * _F)], rows_v.at[b],
                gsem[b]).wait()

        def drain_out(b):
            pltpu.make_async_copy(
                out_v.at[b], out_hbm.at[pl.ds(0, _N)], osem[b]).wait()

        def accum(b):
            def body(n, carry):
                lo = pl.ds(0, _LANES)
                hi = pl.ds(_LANES, _LANES)
                acc0 = rows_v[b, n, lo]
                acc1 = rows_v[b, n, hi]
                for f in range(1, _F):
                    acc0 = acc0 + rows_v[b, f * _N + n, lo]
                    acc1 = acc1 + rows_v[b, f * _N + n, hi]
                out_v[b, n, lo] = acc0
                out_v[b, n, hi] = acc1
                return carry
            lax.fori_loop(0, _N, body, 0)

        fire(0, 0)

        def outer(g, carry):
            for b in range(2):
                ch = g * 2 + b

                @pl.when(ch + 1 < nch)
                def _():
                    fire(ch + 1, 1 - b)

                drain_gather(b)

                @pl.when(ch >= 2)
                def _():
                    drain_out(b)

                accum(b)
                pltpu.async_copy(
                    out_v.at[b],
                    out_hbm.at[pl.ds(w_out_row + ch * _N, _N)],
                    osem[b])
            return carry

        lax.fori_loop(0, nch // 2, outer, 0)
        drain_out(0)
        drain_out(1)

    return sc_kernel


def kernel(phase, tables):
    phase = phase.astype(jnp.int32)
    b_, l_ = phase.shape[0], phase.shape[1]
    rows_total = b_ * l_
    phase2d = phase.reshape(rows_total * _F // 128, 128)
    out = _build(rows_total)(phase2d, tables)
    return out.reshape(b_, l_, _D)


# speedup vs baseline: 12.7950x; 1.0010x over previous
import functools

import jax
import jax.numpy as jnp
from jax import lax
from jax.experimental import pallas as pl
from jax.experimental.pallas import tpu as pltpu
from jax.experimental.pallas import tpu_sc as plsc

_F, _V, _D = 26, 100000, 32
_N = 64
_G = (_N * _F) // 128
_LANES = 16


def _worker_count():
    try:
        info = plsc.get_sparse_core_info()
        return info.num_cores, info.num_subcores
    except Exception:
        return 2, 16


@functools.lru_cache(maxsize=None)
def _build(rows_total):
    nc, ns = _worker_count()
    nw = nc * ns
    rows_per_w = rows_total // nw
    nch = rows_per_w // _N

    mesh = plsc.VectorSubcoreMesh(core_axis_name="c", subcore_axis_name="s")

    @functools.partial(
        pl.kernel,
        mesh=mesh,
        compiler_params=pltpu.CompilerParams(
            use_tc_tiling_on_sc=False, needs_layout_passes=False),
        out_type=jax.ShapeDtypeStruct((rows_total, _D), jnp.float32),
        scratch_types=[
            pltpu.VMEM((2, _G, 128), jnp.int32),
            pltpu.VMEM((2, _F, _N), jnp.int32),
            pltpu.VMEM((2, _F * _N, _D), jnp.float32),
            pltpu.VMEM((2, _N, _D), jnp.float32),
            pltpu.SemaphoreType.DMA,
            pltpu.SemaphoreType.DMA,
            pltpu.SemaphoreType.DMA,
            pltpu.SemaphoreType.DMA,
        ],
    )
    def sc_kernel(phase_hbm, tab_hbm, out_hbm,
                  idx_v, pf_v, rows_v, out_v, g0, g1, s0, s1):
        gsem = (g0, g1)
        osem = (s0, s1)
        wid = lax.axis_index("s") * nc + lax.axis_index("c")
        w_idx_row = wid * (nch * _G)
        w_out_row = wid * rows_per_w

        lane = lax.iota(jnp.int32, _LANES)

        def fire(ch, b):
            pltpu.sync_copy(
                phase_hbm.at[pl.ds(w_idx_row + ch * _G, _G)], idx_v.at[b])
            for k in range(_N // _LANES):
                tok26 = (lane + k * _LANES) * _F
                for f in range(_F):
                    pos = tok26 + f
                    row = lax.shift_right_logical(pos, 7)
                    col = lax.bitwise_and(pos, 127)
                    vals = plsc.load_gather(idx_v.at[b], [row, col])
                    pf_v[b, f, pl.ds(k * _LANES, _LANES)] = vals
            for f in range(_F):
                pltpu.async_copy(
                    tab_hbm.at[f].at[pf_v.at[b, f]],
                    rows_v.at[b, pl.ds(f * _N, _N)],
                    gsem[b])

        def drain_gather(b):
            pltpu.make_async_copy(
                tab_hbm.at[0].at[pl.ds(0, _N * _F)], rows_v.at[b],
                gsem[b]).wait()

        def drain_out(b):
            pltpu.make_async_copy(
                out_v.at[b], out_hbm.at[pl.ds(0, _N)], osem[b]).wait()

        def accum(b):
            def body(n, carry):
                lo = pl.ds(0, _LANES)
                hi = pl.ds(_LANES, _LANES)
                acc0 = rows_v[b, n, lo]
                acc1 = rows_v[b, n, hi]
                for f in range(1, _F):
                    acc0 = acc0 + rows_v[b, f * _N + n, lo]
                    acc1 = acc1 + rows_v[b, f * _N + n, hi]
                out_v[b, n, lo] = acc0
                out_v[b, n, hi] = acc1
                return carry
            lax.fori_loop(0, _N, body, 0)

        fire(0, 0)

        def outer(g, carry):
            for b in range(2):
                ch = g * 2 + b

                @pl.when(ch + 1 < nch)
                def _():
                    fire(ch + 1, 1 - b)

                drain_gather(b)

                @pl.when(ch >= 2)
                def _():
                    drain_out(b)

                accum(b)
                pltpu.async_copy(
                    out_v.at[b],
                    out_hbm.at[pl.ds(w_out_row + ch * _N, _N)],
                    osem[b])
            return carry

        lax.fori_loop(0, nch // 2, outer, 0)
        drain_out(0)
        drain_out(1)

    return sc_kernel


def kernel(phase, tables):
    from jax.experimental.layout import Layout
    from jax.experimental.layout import with_layout_constraint as layout_constraint
    phase = phase.astype(jnp.int32)
    b_, l_ = phase.shape[0], phase.shape[1]
    rows_total = b_ * l_
    tables_lin = layout_constraint(
        tables, Layout((2, 1, 0), tiling=((8,),)))
    phase_lin = layout_constraint(
        phase, Layout((2, 1, 0), tiling=((8,),)))
    phase2d = phase_lin.reshape(rows_total * _F // 128, 128)
    out = _build(rows_total)(phase2d, tables_lin)
    return out.reshape(b_, l_, _D)
